# baseline (speedup 1.0000x reference)
import functools

import jax
import jax.numpy as jnp
from jax import lax
from jax.experimental import pallas as pl
from jax.experimental.pallas import tpu as pltpu

F32 = jnp.float32


def _split4(v, H, W, B, C):
    v6 = v.reshape(H // 2, 2, W // 2, 2, B, C)
    return [v6[:, sy, :, sx, :, :] for sy in range(2) for sx in range(2)]


def _stage_pad(pad_ref, groups, H2, W2, relu):
    for p in range(4):
        sy, sx = divmod(p, 2)
        v = groups[p]
        if relu:
            v = jnp.maximum(v, 0.0)
        oy, ox = sy + 1, sx + 1
        pad_ref[(oy & 1) * 2 + (ox & 1),
                (oy >> 1):(oy >> 1) + H2,
                (ox >> 1):(ox >> 1) + W2, :, :] = v


def _dw3x3(pad_ref, dw_ref, flat_ref, H2, W2, B, C, r=None):
    Mseg = H2 * W2 * B
    for p in range(4):
        sy, sx = divmod(p, 2)
        acc = None
        for dy in range(3):
            for dx in range(3):
                oy, ox = sy + dy, sx + dx
                xq = pad_ref[(oy & 1) * 2 + (ox & 1),
                             (oy >> 1):(oy >> 1) + H2,
                             (ox >> 1):(ox >> 1) + W2, :, :]
                if r is None:
                    coef = dw_ref[dy, dx:dx + 1, :]
                else:
                    coef = dw_ref[0, r, dy, dx:dx + 1, :]
                term = xq * coef
                acc = term if acc is None else acc + term
        flat_ref[p * Mseg:(p + 1) * Mseg, :] = acc.reshape(Mseg, C)


def _xblock(x_groups, dw0, pw0, dw1, pw1, c22, skw, bn,
            pad_a, pad_b, flat_a, flat_b, H2, W2, B, Cin, Cout, swr):
    Mseg = H2 * W2 * B
    pad_a[...] = jnp.zeros_like(pad_a)
    _stage_pad(pad_a, x_groups, H2, W2, relu=swr)
    _dw3x3(pad_a, dw0, flat_a, H2, W2, B, Cin)
    y = jnp.dot(flat_a[...], pw0[...], preferred_element_type=F32)
    y = y * bn[0:1, :] + bn[1:2, :]
    pad_b[...] = jnp.zeros_like(pad_b)
    yg = [y[p * Mseg:(p + 1) * Mseg].reshape(H2, W2, B, Cout)
          for p in range(4)]
    _stage_pad(pad_b, yg, H2, W2, relu=True)
    _dw3x3(pad_b, dw1, flat_b, H2, W2, B, Cout)
    y = jnp.dot(flat_b[...], pw1[...], preferred_element_type=F32)
    y = y * bn[2:3, :] + bn[3:4, :]
    out = None
    for p in range(4):
        ky, kx = divmod(p, 2)
        g = jnp.dot(y[p * Mseg:(p + 1) * Mseg], c22[ky, kx],
                    preferred_element_type=F32)
        out = g if out is None else out + g
    out = out + bn[4:5, :]
    skip = jnp.dot(x_groups[0].reshape(Mseg, Cin), skw[0, 0],
                   preferred_element_type=F32)
    out = out + (skip * bn[5:6, :] + bn[6:7, :])
    return out


def _enc_kernel(xcol_ref, w1_ref, sv_ref, w2_ref,
                b0dw0, b0pw0, b0dw1, b0pw1, b0c22, b0skw, b0bn,
                b1dw0, b1pw0, b1dw1, b1pw1, b1c22, b1skw, b1bn,
                b2dw0, b2pw0, b2dw1, b2pw1, b2c22, b2skw, b2bn,
                mdw_ref, mpw0_ref, mpw1_ref, mpw2_ref, msc_ref, msh_ref,
                o_ref,
                spad, p0a, p0b, f0a, f0b, p1a, p1b, f1a, f1b,
                p2a, p2b, f2a, f2b, mflat, *, B, nmid):
    k = pl.program_id(0)

    @pl.when(k == 0)
    def _front_step():
        _front_body(xcol_ref, w1_ref, sv_ref, w2_ref,
                    b0dw0, b0pw0, b0dw1, b0pw1, b0c22, b0skw, b0bn,
                    b1dw0, b1pw0, b1dw1, b1pw1, b1c22, b1skw, b1bn,
                    b2dw0, b2pw0, b2dw1, b2pw1, b2c22, b2skw, b2bn,
                    o_ref, spad, p0a, p0b, f0a, f0b, p1a, p1b, f1a, f1b,
                    p2a, p2b, f2a, f2b, B)

    @pl.when(k > 0)
    def _mid_step():
        C = o_ref.shape[-1]
        x0 = o_ref[...]
        t = x0
        for r in range(3):
            xr = jnp.maximum(t, 0.0)
            for p in range(4):
                h, w = divmod(p, 2)
                acc = None
                for q in range(4):
                    i, j = divmod(q, 2)
                    coef = mdw_ref[0, r, 1 + i - h, 1 + j - w:2 + j - w, :]
                    term = xr[q] * coef
                    acc = term if acc is None else acc + term
                mflat[p * B:(p + 1) * B, :] = acc
            pw = (mpw0_ref, mpw1_ref, mpw2_ref)[r][0, 0]
            z = jnp.dot(mflat[...], pw, preferred_element_type=F32)
            z = z * msc_ref[0, r:r + 1, :] + msh_ref[0, r:r + 1, :]
            t = z.reshape(4, B, C)
        out = t + x0
        out = jnp.where(k == nmid, jnp.maximum(out, 0.0), out)
        o_ref[...] = out


def _front_body(xcol_ref, w1_ref, sv_ref, w2_ref,
                b0dw0, b0pw0, b0dw1, b0pw1, b0c22, b0skw, b0bn,
                b1dw0, b1pw0, b1dw1, b1pw1, b1c22, b1skw, b1bn,
                b2dw0, b2pw0, b2dw1, b2pw1, b2c22, b2skw, b2bn,
                o_ref, spad, p0a, p0b, f0a, f0b, p1a, p1b, f1a, f1b,
                p2a, p2b, f2a, f2b, B):
    H = W = 16
    xcolT = xcol_ref[...]
    y1 = lax.dot_general(xcolT, w1_ref[...], (((0,), (0,)), ((), ())),
                         preferred_element_type=F32)
    y1 = jnp.maximum(y1 * sv_ref[0:1, :32] + sv_ref[1:2, :32], 0.0)
    spad[...] = jnp.zeros_like(spad)
    spad[1:H + 1, 1:W + 1, :, :] = y1.reshape(H, W, B, 32)
    acc = None
    for t in range(9):
        ky, kx = divmod(t, 3)
        xt = spad[ky:ky + H, kx:kx + W, :, :].reshape(H * W * B, 32)
        g = jnp.dot(xt, w2_ref[ky, kx], preferred_element_type=F32)
        acc = g if acc is None else acc + g
    y2 = jnp.maximum(acc * sv_ref[2:3, :] + sv_ref[3:4, :], 0.0)

    g0 = _split4(y2.reshape(H, W, B, 64), H, W, B, 64)
    o0 = _xblock(g0, b0dw0, b0pw0, b0dw1, b0pw1, b0c22, b0skw, b0bn,
                 p0a, p0b, f0a, f0b, 8, 8, B, 64, 128, swr=False)
    g1 = _split4(o0.reshape(8, 8, B, 128), 8, 8, B, 128)
    o1 = _xblock(g1, b1dw0, b1pw0, b1dw1, b1pw1, b1c22, b1skw, b1bn,
                 p1a, p1b, f1a, f1b, 4, 4, B, 128, 256, swr=True)
    g2 = _split4(o1.reshape(4, 4, B, 256), 4, 4, B, 256)
    o2 = _xblock(g2, b2dw0, b2pw0, b2dw1, b2pw1, b2c22, b2skw, b2bn,
                 p2a, p2b, f2a, f2b, 2, 2, B, 256, 728, swr=True)
    o_ref[...] = o2.reshape(4, B, 728)


def _encoder(x_nchw, w1, s1, t1, w2, s2, t2, bw,
             mid_dw, mid_pw, mid_sc, mid_sh):
    nmid = mid_dw.shape[0]
    B = x_nchw.shape[0]
    xcol = lax.conv_general_dilated_patches(
        x_nchw.astype(F32), filter_shape=(3, 3), window_strides=(2, 2),
        padding=((1, 1), (1, 1)),
        dimension_numbers=("NCHW", "OIHW", "CHWN"))
    xcol = xcol.reshape(27, 16 * 16 * B)
    w1r = jnp.transpose(w1, (2, 0, 1, 3))

    def cs(shape):
        nd = len(shape)
        return pl.BlockSpec(shape, lambda i, _n=nd: (0,) * _n)

    sv = jnp.stack([jnp.pad(s1, (0, 32)), jnp.pad(t1, (0, 32)), s2, t2])
    inputs = [xcol, w1r.reshape(27, 32), sv, w2]
    in_specs = [cs((27, 16 * 16 * B)),
                cs((27, 32)), cs((4, 64)), cs((3, 3, 32, 64))]
    for bi, (Cin, Cout) in enumerate(((64, 128), (128, 256), (256, 728))):
        (dw0, pw0, sc0, sh0), (dw1, pw1, sc1, sh1), c22w, c22b, skw, sksc, \
            sksh = bw[bi]
        bn = jnp.stack([sc0, sh0, sc1, sh1, c22b, sksc, sksh])
        inputs += [dw0, pw0, dw1, pw1, c22w, skw, bn]
        in_specs += [cs((3, 3, Cin)), cs((Cin, Cout)),
                     cs((3, 3, Cout)), cs((Cout, Cout)),
                     cs((2, 2, Cout, Cout)), cs((1, 1, Cin, Cout)),
                     cs((7, Cout))]

    C = mid_pw.shape[-1]

    def mid_map(k, *rest):
        return (jnp.maximum(k - 1, 0),) + rest

    inputs += [mid_dw, mid_pw, mid_pw, mid_pw, mid_sc, mid_sh]
    in_specs += [
        pl.BlockSpec((1, 3, 3, 3, C), lambda k: mid_map(k, 0, 0, 0, 0)),
        pl.BlockSpec((1, 1, C, C), lambda k: mid_map(k, 0, 0, 0)),
        pl.BlockSpec((1, 1, C, C), lambda k: mid_map(k, 1, 0, 0)),
        pl.BlockSpec((1, 1, C, C), lambda k: mid_map(k, 2, 0, 0)),
        pl.BlockSpec((1, 3, C), lambda k: mid_map(k, 0, 0)),
        pl.BlockSpec((1, 3, C), lambda k: mid_map(k, 0, 0)),
    ]

    def pads(H2, C):
        return pltpu.VMEM((4, H2 + 1, H2 + 1, B, C), F32)

    def flats(H2, C):
        return pltpu.VMEM((4 * H2 * H2 * B, C), F32)

    scratch = [pltpu.VMEM((18, 18, B, 32), F32),
               pads(8, 64), pads(8, 128), flats(8, 64), flats(8, 128),
               pads(4, 128), pads(4, 256), flats(4, 128), flats(4, 256),
               pads(2, 256), pads(2, 728), flats(2, 256), flats(2, 728),
               pltpu.VMEM((4 * B, C), F32)]

    return pl.pallas_call(
        functools.partial(_enc_kernel, B=B, nmid=nmid),
        out_shape=jax.ShapeDtypeStruct((4, B, 728), F32),
        grid=(1 + nmid,),
        in_specs=in_specs,
        out_specs=pl.BlockSpec((4, B, 728), lambda k: (0, 0, 0)),
        scratch_shapes=scratch,
        compiler_params=pltpu.CompilerParams(
            dimension_semantics=("arbitrary",)),
    )(*inputs)


def kernel(x, conv1_w, bn1_scale, bn1_shift, conv2_w, bn2_scale, bn2_shift,
           b0_sep0_dw, b0_sep0_pw, b0_sep0_sc, b0_sep0_sh,
           b0_sep1_dw, b0_sep1_pw, b0_sep1_sc, b0_sep1_sh,
           b0_c22w, b0_c22b, b0_skw, b0_sksc, b0_sksh,
           b1_sep0_dw, b1_sep0_pw, b1_sep0_sc, b1_sep0_sh,
           b1_sep1_dw, b1_sep1_pw, b1_sep1_sc, b1_sep1_sh,
           b1_c22w, b1_c22b, b1_skw, b1_sksc, b1_sksh,
           b2_sep0_dw, b2_sep0_pw, b2_sep0_sc, b2_sep0_sh,
           b2_sep1_dw, b2_sep1_pw, b2_sep1_sc, b2_sep1_sh,
           b2_c22w, b2_c22b, b2_skw, b2_sksc, b2_sksh,
           mid_dw, mid_pw, mid_sc, mid_sh):
    B = x.shape[0]
    bw = (
        ((b0_sep0_dw, b0_sep0_pw, b0_sep0_sc, b0_sep0_sh),
         (b0_sep1_dw, b0_sep1_pw, b0_sep1_sc, b0_sep1_sh),
         b0_c22w, b0_c22b, b0_skw, b0_sksc, b0_sksh),
        ((b1_sep0_dw, b1_sep0_pw, b1_sep0_sc, b1_sep0_sh),
         (b1_sep1_dw, b1_sep1_pw, b1_sep1_sc, b1_sep1_sh),
         b1_c22w, b1_c22b, b1_skw, b1_sksc, b1_sksh),
        ((b2_sep0_dw, b2_sep0_pw, b2_sep0_sc, b2_sep0_sh),
         (b2_sep1_dw, b2_sep1_pw, b2_sep1_sc, b2_sep1_sh),
         b2_c22w, b2_c22b, b2_skw, b2_sksc, b2_sksh),
    )
    y = _encoder(x, conv1_w, bn1_scale, bn1_shift,
                 conv2_w, bn2_scale, bn2_shift, bw,
                 mid_dw, mid_pw, mid_sc, mid_sh)
    C = y.shape[-1]
    y = y.reshape(2, 2, B, C)
    return jnp.transpose(y, (2, 3, 0, 1))


# speedup vs baseline: 1.0022x; 1.0022x over previous
import functools

import jax
import jax.numpy as jnp
from jax import lax
from jax.experimental import pallas as pl
from jax.experimental.pallas import tpu as pltpu

F32 = jnp.float32


def _split4(v, H, W, B, C):
    v6 = v.reshape(H // 2, 2, W // 2, 2, B, C)
    return [v6[:, sy, :, sx, :, :] for sy in range(2) for sx in range(2)]


def _stage_pad(pad_ref, groups, H2, W2, relu):
    for p in range(4):
        sy, sx = divmod(p, 2)
        v = groups[p]
        if relu:
            v = jnp.maximum(v, 0.0)
        oy, ox = sy + 1, sx + 1
        pad_ref[(oy & 1) * 2 + (ox & 1),
                (oy >> 1):(oy >> 1) + H2,
                (ox >> 1):(ox >> 1) + W2, :, :] = v


def _dw3x3(pad_ref, dw_ref, flat_ref, H2, W2, B, C, r=None):
    Mseg = H2 * W2 * B
    for p in range(4):
        sy, sx = divmod(p, 2)
        acc = None
        for dy in range(3):
            for dx in range(3):
                oy, ox = sy + dy, sx + dx
                xq = pad_ref[(oy & 1) * 2 + (ox & 1),
                             (oy >> 1):(oy >> 1) + H2,
                             (ox >> 1):(ox >> 1) + W2, :, :]
                if r is None:
                    coef = dw_ref[dy, dx:dx + 1, :]
                else:
                    coef = dw_ref[0, r, dy, dx:dx + 1, :]
                term = xq * coef
                acc = term if acc is None else acc + term
        flat_ref[p * Mseg:(p + 1) * Mseg, :] = acc.reshape(Mseg, C)


def _xblock(x_groups, dw0, pw0, dw1, pw1, c22, skw, bn,
            pad_a, pad_b, flat_a, flat_b, H2, W2, B, Cin, Cout, swr):
    Mseg = H2 * W2 * B
    pad_a[...] = jnp.zeros_like(pad_a)
    _stage_pad(pad_a, x_groups, H2, W2, relu=swr)
    _dw3x3(pad_a, dw0, flat_a, H2, W2, B, Cin)
    y = jnp.dot(flat_a[...], pw0[...], preferred_element_type=F32)
    y = y * bn[0:1, :] + bn[1:2, :]
    pad_b[...] = jnp.zeros_like(pad_b)
    yg = [y[p * Mseg:(p + 1) * Mseg].reshape(H2, W2, B, Cout)
          for p in range(4)]
    _stage_pad(pad_b, yg, H2, W2, relu=True)
    _dw3x3(pad_b, dw1, flat_b, H2, W2, B, Cout)
    y = jnp.dot(flat_b[...], pw1[...], preferred_element_type=F32)
    y = y * bn[2:3, :] + bn[3:4, :]
    out = None
    for p in range(4):
        ky, kx = divmod(p, 2)
        g = jnp.dot(y[p * Mseg:(p + 1) * Mseg], c22[ky, kx],
                    preferred_element_type=F32)
        out = g if out is None else out + g
    out = out + bn[4:5, :]
    skip = jnp.dot(x_groups[0].reshape(Mseg, Cin), skw[0, 0],
                   preferred_element_type=F32)
    out = out + (skip * bn[5:6, :] + bn[6:7, :])
    return out


def _enc_kernel(xcol_ref, w1_ref, sv_ref, w2_ref,
                b0dw0, b0pw0, b0dw1, b0pw1, b0c22, b0skw, b0bn,
                b1dw0, b1pw0, b1dw1, b1pw1, b1c22, b1skw, b1bn,
                b2dw0, b2pw0, b2dw1, b2pw1, b2c22, b2skw, b2bn,
                mdw_ref, mpw0_ref, msc_ref, msh_ref,
                o_ref,
                spad, p0a, p0b, f0a, f0b, p1a, p1b, f1a, f1b,
                p2a, p2b, f2a, f2b, mflat, *, B, nmid):
    k = pl.program_id(0)

    @pl.when(k == 0)
    def _front_step():
        _front_body(xcol_ref, w1_ref, sv_ref, w2_ref,
                    b0dw0, b0pw0, b0dw1, b0pw1, b0c22, b0skw, b0bn,
                    b1dw0, b1pw0, b1dw1, b1pw1, b1c22, b1skw, b1bn,
                    b2dw0, b2pw0, b2dw1, b2pw1, b2c22, b2skw, b2bn,
                    o_ref, spad, p0a, p0b, f0a, f0b, p1a, p1b, f1a, f1b,
                    p2a, p2b, f2a, f2b, B)

    @pl.when(k > 0)
    def _mid_step():
        C = o_ref.shape[-1]
        x0 = o_ref[...]
        t = x0
        for r in range(3):
            xr = jnp.maximum(t, 0.0)
            for p in range(4):
                h, w = divmod(p, 2)
                acc = None
                for q in range(4):
                    i, j = divmod(q, 2)
                    coef = mdw_ref[0, r, 1 + i - h, 1 + j - w:2 + j - w, :]
                    term = xr[q] * coef
                    acc = term if acc is None else acc + term
                mflat[p * B:(p + 1) * B, :] = acc
            pw = mpw0_ref[0, r]
            z = jnp.dot(mflat[...], pw, preferred_element_type=F32)
            z = z * msc_ref[0, r:r + 1, :] + msh_ref[0, r:r + 1, :]
            t = z.reshape(4, B, C)
        out = t + x0
        out = jnp.where(k == nmid, jnp.maximum(out, 0.0), out)
        o_ref[...] = out


def _front_body(xcol_ref, w1_ref, sv_ref, w2_ref,
                b0dw0, b0pw0, b0dw1, b0pw1, b0c22, b0skw, b0bn,
                b1dw0, b1pw0, b1dw1, b1pw1, b1c22, b1skw, b1bn,
                b2dw0, b2pw0, b2dw1, b2pw1, b2c22, b2skw, b2bn,
                o_ref, spad, p0a, p0b, f0a, f0b, p1a, p1b, f1a, f1b,
                p2a, p2b, f2a, f2b, B):
    H = W = 16
    xcolT = xcol_ref[...]
    y1 = lax.dot_general(xcolT, w1_ref[...], (((0,), (0,)), ((), ())),
                         preferred_element_type=F32)
    y1 = jnp.maximum(y1 * sv_ref[0:1, :32] + sv_ref[1:2, :32], 0.0)
    spad[...] = jnp.zeros_like(spad)
    spad[1:H + 1, 1:W + 1, :, :] = y1.reshape(H, W, B, 32)
    acc = None
    for t in range(9):
        ky, kx = divmod(t, 3)
        xt = spad[ky:ky + H, kx:kx + W, :, :].reshape(H * W * B, 32)
        g = jnp.dot(xt, w2_ref[ky, kx], preferred_element_type=F32)
        acc = g if acc is None else acc + g
    y2 = jnp.maximum(acc * sv_ref[2:3, :] + sv_ref[3:4, :], 0.0)

    g0 = _split4(y2.reshape(H, W, B, 64), H, W, B, 64)
    o0 = _xblock(g0, b0dw0, b0pw0, b0dw1, b0pw1, b0c22, b0skw, b0bn,
                 p0a, p0b, f0a, f0b, 8, 8, B, 64, 128, swr=False)
    g1 = _split4(o0.reshape(8, 8, B, 128), 8, 8, B, 128)
    o1 = _xblock(g1, b1dw0, b1pw0, b1dw1, b1pw1, b1c22, b1skw, b1bn,
                 p1a, p1b, f1a, f1b, 4, 4, B, 128, 256, swr=True)
    g2 = _split4(o1.reshape(4, 4, B, 256), 4, 4, B, 256)
    o2 = _xblock(g2, b2dw0, b2pw0, b2dw1, b2pw1, b2c22, b2skw, b2bn,
                 p2a, p2b, f2a, f2b, 2, 2, B, 256, 728, swr=True)
    o_ref[...] = o2.reshape(4, B, 728)


def _encoder(x_nchw, w1, s1, t1, w2, s2, t2, bw,
             mid_dw, mid_pw, mid_sc, mid_sh):
    nmid = mid_dw.shape[0]
    B = x_nchw.shape[0]
    xcol = lax.conv_general_dilated_patches(
        x_nchw.astype(F32), filter_shape=(3, 3), window_strides=(2, 2),
        padding=((1, 1), (1, 1)),
        dimension_numbers=("NCHW", "OIHW", "CHWN"))
    xcol = xcol.reshape(27, 16 * 16 * B)
    w1r = jnp.transpose(w1, (2, 0, 1, 3))

    def cs(shape):
        nd = len(shape)
        return pl.BlockSpec(shape, lambda i, _n=nd: (0,) * _n)

    sv = jnp.stack([jnp.pad(s1, (0, 32)), jnp.pad(t1, (0, 32)), s2, t2])
    inputs = [xcol, w1r.reshape(27, 32), sv, w2]
    in_specs = [cs((27, 16 * 16 * B)),
                cs((27, 32)), cs((4, 64)), cs((3, 3, 32, 64))]
    for bi, (Cin, Cout) in enumerate(((64, 128), (128, 256), (256, 728))):
        (dw0, pw0, sc0, sh0), (dw1, pw1, sc1, sh1), c22w, c22b, skw, sksc, \
            sksh = bw[bi]
        bn = jnp.stack([sc0, sh0, sc1, sh1, c22b, sksc, sksh])
        inputs += [dw0, pw0, dw1, pw1, c22w, skw, bn]
        in_specs += [cs((3, 3, Cin)), cs((Cin, Cout)),
                     cs((3, 3, Cout)), cs((Cout, Cout)),
                     cs((2, 2, Cout, Cout)), cs((1, 1, Cin, Cout)),
                     cs((7, Cout))]

    C = mid_pw.shape[-1]

    def mid_map(k, *rest):
        return (jnp.maximum(k - 1, 0),) + rest

    inputs += [mid_dw, mid_pw, mid_sc, mid_sh]
    in_specs += [
        pl.BlockSpec((1, 3, 3, 3, C), lambda k: mid_map(k, 0, 0, 0, 0)),
        pl.BlockSpec((1, 3, C, C), lambda k: mid_map(k, 0, 0, 0)),
        pl.BlockSpec((1, 3, C), lambda k: mid_map(k, 0, 0)),
        pl.BlockSpec((1, 3, C), lambda k: mid_map(k, 0, 0)),
    ]

    def pads(H2, C):
        return pltpu.VMEM((4, H2 + 1, H2 + 1, B, C), F32)

    def flats(H2, C):
        return pltpu.VMEM((4 * H2 * H2 * B, C), F32)

    scratch = [pltpu.VMEM((18, 18, B, 32), F32),
               pads(8, 64), pads(8, 128), flats(8, 64), flats(8, 128),
               pads(4, 128), pads(4, 256), flats(4, 128), flats(4, 256),
               pads(2, 256), pads(2, 728), flats(2, 256), flats(2, 728),
               pltpu.VMEM((4 * B, C), F32)]

    return pl.pallas_call(
        functools.partial(_enc_kernel, B=B, nmid=nmid),
        out_shape=jax.ShapeDtypeStruct((4, B, 728), F32),
        grid=(1 + nmid,),
        in_specs=in_specs,
        out_specs=pl.BlockSpec((4, B, 728), lambda k: (0, 0, 0)),
        scratch_shapes=scratch,
        compiler_params=pltpu.CompilerParams(
            dimension_semantics=("arbitrary",)),
    )(*inputs)


def kernel(x, conv1_w, bn1_scale, bn1_shift, conv2_w, bn2_scale, bn2_shift,
           b0_sep0_dw, b0_sep0_pw, b0_sep0_sc, b0_sep0_sh,
           b0_sep1_dw, b0_sep1_pw, b0_sep1_sc, b0_sep1_sh,
           b0_c22w, b0_c22b, b0_skw, b0_sksc, b0_sksh,
           b1_sep0_dw, b1_sep0_pw, b1_sep0_sc, b1_sep0_sh,
           b1_sep1_dw, b1_sep1_pw, b1_sep1_sc, b1_sep1_sh,
           b1_c22w, b1_c22b, b1_skw, b1_sksc, b1_sksh,
           b2_sep0_dw, b2_sep0_pw, b2_sep0_sc, b2_sep0_sh,
           b2_sep1_dw, b2_sep1_pw, b2_sep1_sc, b2_sep1_sh,
           b2_c22w, b2_c22b, b2_skw, b2_sksc, b2_sksh,
           mid_dw, mid_pw, mid_sc, mid_sh):
    B = x.shape[0]
    bw = (
        ((b0_sep0_dw, b0_sep0_pw, b0_sep0_sc, b0_sep0_sh),
         (b0_sep1_dw, b0_sep1_pw, b0_sep1_sc, b0_sep1_sh),
         b0_c22w, b0_c22b, b0_skw, b0_sksc, b0_sksh),
        ((b1_sep0_dw, b1_sep0_pw, b1_sep0_sc, b1_sep0_sh),
         (b1_sep1_dw, b1_sep1_pw, b1_sep1_sc, b1_sep1_sh),
         b1_c22w, b1_c22b, b1_skw, b1_sksc, b1_sksh),
        ((b2_sep0_dw, b2_sep0_pw, b2_sep0_sc, b2_sep0_sh),
         (b2_sep1_dw, b2_sep1_pw, b2_sep1_sc, b2_sep1_sh),
         b2_c22w, b2_c22b, b2_skw, b2_sksc, b2_sksh),
    )
    y = _encoder(x, conv1_w, bn1_scale, bn1_shift,
                 conv2_w, bn2_scale, bn2_shift, bw,
                 mid_dw, mid_pw, mid_sc, mid_sh)
    C = y.shape[-1]
    y = y.reshape(2, 2, B, C)
    return jnp.transpose(y, (2, 3, 0, 1))


# speedup vs baseline: 1.0699x; 1.0675x over previous
import functools

import jax
import jax.numpy as jnp
from jax import lax
from jax.experimental import pallas as pl
from jax.experimental.pallas import tpu as pltpu

F32 = jnp.float32


def _split4(v, H, W, B, C):
    v6 = v.reshape(H // 2, 2, W // 2, 2, B, C)
    return [v6[:, sy, :, sx, :, :] for sy in range(2) for sx in range(2)]


def _stage_pad(pad_ref, groups, H2, W2, relu):
    for p in range(4):
        sy, sx = divmod(p, 2)
        v = groups[p]
        if relu:
            v = jnp.maximum(v, 0.0)
        oy, ox = sy + 1, sx + 1
        pad_ref[(oy & 1) * 2 + (ox & 1),
                (oy >> 1):(oy >> 1) + H2,
                (ox >> 1):(ox >> 1) + W2, :, :] = v


def _dw3x3(pad_ref, dw_ref, flat_ref, H2, W2, B, C, r=None):
    Mseg = H2 * W2 * B
    for p in range(4):
        sy, sx = divmod(p, 2)
        acc = None
        for dy in range(3):
            for dx in range(3):
                oy, ox = sy + dy, sx + dx
                xq = pad_ref[(oy & 1) * 2 + (ox & 1),
                             (oy >> 1):(oy >> 1) + H2,
                             (ox >> 1):(ox >> 1) + W2, :, :]
                if r is None:
                    coef = dw_ref[dy, dx:dx + 1, :]
                else:
                    coef = dw_ref[0, r, dy, dx:dx + 1, :]
                term = xq * coef
                acc = term if acc is None else acc + term
        flat_ref[p * Mseg:(p + 1) * Mseg, :] = acc.reshape(Mseg, C)


def _xblock(x_groups, dw0, pw0, dw1, pw1, c22, skw, bn,
            pad_a, pad_b, flat_a, flat_b, H2, W2, B, Cin, Cout, swr):
    Mseg = H2 * W2 * B
    pad_a[...] = jnp.zeros_like(pad_a)
    _stage_pad(pad_a, x_groups, H2, W2, relu=swr)
    _dw3x3(pad_a, dw0, flat_a, H2, W2, B, Cin)
    y = jnp.dot(flat_a[...], pw0[...], preferred_element_type=F32)
    y = y * bn[0:1, :] + bn[1:2, :]
    pad_b[...] = jnp.zeros_like(pad_b)
    yg = [y[p * Mseg:(p + 1) * Mseg].reshape(H2, W2, B, Cout)
          for p in range(4)]
    _stage_pad(pad_b, yg, H2, W2, relu=True)
    _dw3x3(pad_b, dw1, flat_b, H2, W2, B, Cout)
    y = jnp.dot(flat_b[...], pw1[...], preferred_element_type=F32)
    y = y * bn[2:3, :] + bn[3:4, :]
    out = None
    for p in range(4):
        ky, kx = divmod(p, 2)
        g = jnp.dot(y[p * Mseg:(p + 1) * Mseg], c22[ky, kx],
                    preferred_element_type=F32)
        out = g if out is None else out + g
    out = out + bn[4:5, :]
    skip = jnp.dot(x_groups[0].reshape(Mseg, Cin), skw[0, 0],
                   preferred_element_type=F32)
    out = out + (skip * bn[5:6, :] + bn[6:7, :])
    return out


def _enc_kernel(xcol_ref, w1_ref, sv_ref, w2_ref,
                b0dw0, b0pw0, b0dw1, b0pw1, b0c22, b0skw, b0bn,
                b1dw0, b1pw0, b1dw1, b1pw1, b1c22, b1skw, b1bn,
                b2dw0, b2pw0, b2dw1, b2pw1, b2c22, b2skw, b2bn,
                mdw_ref, mpw0_ref, msc_ref, msh_ref,
                o_ref,
                spad, p0a, p0b, f0a, f0b, p1a, p1b, f1a, f1b,
                p2a, p2b, f2a, f2b, mflat,
                b2pw0_v, b2pw1_v, b2c22_v, b2skw_v, dsem, *, B, nmid):
    k = pl.program_id(0)

    @pl.when(k == 0)
    def _front_step():
        cps = [pltpu.make_async_copy(b2pw0, b2pw0_v, dsem.at[0]),
               pltpu.make_async_copy(b2pw1, b2pw1_v, dsem.at[1]),
               pltpu.make_async_copy(b2c22, b2c22_v, dsem.at[2]),
               pltpu.make_async_copy(b2skw, b2skw_v, dsem.at[3])]
        for c in cps:
            c.start()

        def wait_b2():
            for c in cps:
                c.wait()

        _front_body(xcol_ref, w1_ref, sv_ref, w2_ref,
                    b0dw0, b0pw0, b0dw1, b0pw1, b0c22, b0skw, b0bn,
                    b1dw0, b1pw0, b1dw1, b1pw1, b1c22, b1skw, b1bn,
                    b2dw0, b2pw0_v, b2dw1, b2pw1_v, b2c22_v, b2skw_v, b2bn,
                    o_ref, spad, p0a, p0b, f0a, f0b, p1a, p1b, f1a, f1b,
                    p2a, p2b, f2a, f2b, B, wait_b2)

    @pl.when(k > 0)
    def _mid_step():
        C = o_ref.shape[-1]
        x0 = o_ref[...]
        t = x0
        for r in range(3):
            xr = jnp.maximum(t, 0.0)
            for p in range(4):
                h, w = divmod(p, 2)
                acc = None
                for q in range(4):
                    i, j = divmod(q, 2)
                    coef = mdw_ref[0, r, 1 + i - h, 1 + j - w:2 + j - w, :]
                    term = xr[q] * coef
                    acc = term if acc is None else acc + term
                mflat[p * B:(p + 1) * B, :] = acc
            pw = mpw0_ref[0, r]
            z = jnp.dot(mflat[...], pw, preferred_element_type=F32)
            z = z * msc_ref[0, r:r + 1, :] + msh_ref[0, r:r + 1, :]
            t = z.reshape(4, B, C)
        out = t + x0
        out = jnp.where(k == nmid, jnp.maximum(out, 0.0), out)
        o_ref[...] = out


def _front_body(xcol_ref, w1_ref, sv_ref, w2_ref,
                b0dw0, b0pw0, b0dw1, b0pw1, b0c22, b0skw, b0bn,
                b1dw0, b1pw0, b1dw1, b1pw1, b1c22, b1skw, b1bn,
                b2dw0, b2pw0, b2dw1, b2pw1, b2c22, b2skw, b2bn,
                o_ref, spad, p0a, p0b, f0a, f0b, p1a, p1b, f1a, f1b,
                p2a, p2b, f2a, f2b, B, wait_b2=None):
    H = W = 16
    xcolT = xcol_ref[...]
    y1 = lax.dot_general(xcolT, w1_ref[...], (((0,), (0,)), ((), ())),
                         preferred_element_type=F32)
    y1 = jnp.maximum(y1 * sv_ref[0:1, :32] + sv_ref[1:2, :32], 0.0)
    spad[...] = jnp.zeros_like(spad)
    spad[1:H + 1, 1:W + 1, :, :] = y1.reshape(H, W, B, 32)
    acc = None
    for t in range(9):
        ky, kx = divmod(t, 3)
        xt = spad[ky:ky + H, kx:kx + W, :, :].reshape(H * W * B, 32)
        g = jnp.dot(xt, w2_ref[ky, kx], preferred_element_type=F32)
        acc = g if acc is None else acc + g
    y2 = jnp.maximum(acc * sv_ref[2:3, :] + sv_ref[3:4, :], 0.0)

    g0 = _split4(y2.reshape(H, W, B, 64), H, W, B, 64)
    o0 = _xblock(g0, b0dw0, b0pw0, b0dw1, b0pw1, b0c22, b0skw, b0bn,
                 p0a, p0b, f0a, f0b, 8, 8, B, 64, 128, swr=False)
    g1 = _split4(o0.reshape(8, 8, B, 128), 8, 8, B, 128)
    o1 = _xblock(g1, b1dw0, b1pw0, b1dw1, b1pw1, b1c22, b1skw, b1bn,
                 p1a, p1b, f1a, f1b, 4, 4, B, 128, 256, swr=True)
    g2 = _split4(o1.reshape(4, 4, B, 256), 4, 4, B, 256)
    if wait_b2 is not None:
        wait_b2()
    o2 = _xblock(g2, b2dw0, b2pw0, b2dw1, b2pw1, b2c22, b2skw, b2bn,
                 p2a, p2b, f2a, f2b, 2, 2, B, 256, 728, swr=True)
    o_ref[...] = o2.reshape(4, B, 728)


def _encoder(x_nchw, w1, s1, t1, w2, s2, t2, bw,
             mid_dw, mid_pw, mid_sc, mid_sh):
    nmid = mid_dw.shape[0]
    B = x_nchw.shape[0]
    xcol = lax.conv_general_dilated_patches(
        x_nchw.astype(F32), filter_shape=(3, 3), window_strides=(2, 2),
        padding=((1, 1), (1, 1)),
        dimension_numbers=("NCHW", "OIHW", "CHWN"))
    xcol = xcol.reshape(27, 16 * 16 * B)
    w1r = jnp.transpose(w1, (2, 0, 1, 3))

    def cs(shape):
        nd = len(shape)
        return pl.BlockSpec(shape, lambda i, _n=nd: (0,) * _n)

    sv = jnp.stack([jnp.pad(s1, (0, 32)), jnp.pad(t1, (0, 32)), s2, t2])
    inputs = [xcol, w1r.reshape(27, 32), sv, w2]
    in_specs = [cs((27, 16 * 16 * B)),
                cs((27, 32)), cs((4, 64)), cs((3, 3, 32, 64))]
    for bi, (Cin, Cout) in enumerate(((64, 128), (128, 256), (256, 728))):
        (dw0, pw0, sc0, sh0), (dw1, pw1, sc1, sh1), c22w, c22b, skw, sksc, \
            sksh = bw[bi]
        bn = jnp.stack([sc0, sh0, sc1, sh1, c22b, sksc, sksh])
        inputs += [dw0, pw0, dw1, pw1, c22w, skw, bn]
        if bi == 2:
            hbm = pl.BlockSpec(memory_space=pl.ANY)
            in_specs += [cs((3, 3, Cin)), hbm, cs((3, 3, Cout)), hbm,
                         hbm, hbm, cs((7, Cout))]
        else:
            in_specs += [cs((3, 3, Cin)), cs((Cin, Cout)),
                         cs((3, 3, Cout)), cs((Cout, Cout)),
                         cs((2, 2, Cout, Cout)), cs((1, 1, Cin, Cout)),
                         cs((7, Cout))]

    C = mid_pw.shape[-1]

    def mid_map(k, *rest):
        return (jnp.maximum(k - 1, 0),) + rest

    inputs += [mid_dw, mid_pw, mid_sc, mid_sh]
    in_specs += [
        pl.BlockSpec((1, 3, 3, 3, C), lambda k: mid_map(k, 0, 0, 0, 0)),
        pl.BlockSpec((1, 3, C, C), lambda k: mid_map(k, 0, 0, 0)),
        pl.BlockSpec((1, 3, C), lambda k: mid_map(k, 0, 0)),
        pl.BlockSpec((1, 3, C), lambda k: mid_map(k, 0, 0)),
    ]

    def pads(H2, C):
        return pltpu.VMEM((4, H2 + 1, H2 + 1, B, C), F32)

    def flats(H2, C):
        return pltpu.VMEM((4 * H2 * H2 * B, C), F32)

    scratch = [pltpu.VMEM((18, 18, B, 32), F32),
               pads(8, 64), pads(8, 128), flats(8, 64), flats(8, 128),
               pads(4, 128), pads(4, 256), flats(4, 128), flats(4, 256),
               pads(2, 256), pads(2, 728), flats(2, 256), flats(2, 728),
               pltpu.VMEM((4 * B, C), F32),
               pltpu.VMEM((256, 728), F32), pltpu.VMEM((728, 728), F32),
               pltpu.VMEM((2, 2, 728, 728), F32),
               pltpu.VMEM((1, 1, 256, 728), F32),
               pltpu.SemaphoreType.DMA((4,))]

    return pl.pallas_call(
        functools.partial(_enc_kernel, B=B, nmid=nmid),
        out_shape=jax.ShapeDtypeStruct((4, B, 728), F32),
        grid=(1 + nmid,),
        in_specs=in_specs,
        out_specs=pl.BlockSpec((4, B, 728), lambda k: (0, 0, 0)),
        scratch_shapes=scratch,
        compiler_params=pltpu.CompilerParams(
            dimension_semantics=("arbitrary",)),
    )(*inputs)


def kernel(x, conv1_w, bn1_scale, bn1_shift, conv2_w, bn2_scale, bn2_shift,
           b0_sep0_dw, b0_sep0_pw, b0_sep0_sc, b0_sep0_sh,
           b0_sep1_dw, b0_sep1_pw, b0_sep1_sc, b0_sep1_sh,
           b0_c22w, b0_c22b, b0_skw, b0_sksc, b0_sksh,
           b1_sep0_dw, b1_sep0_pw, b1_sep0_sc, b1_sep0_sh,
           b1_sep1_dw, b1_sep1_pw, b1_sep1_sc, b1_sep1_sh,
           b1_c22w, b1_c22b, b1_skw, b1_sksc, b1_sksh,
           b2_sep0_dw, b2_sep0_pw, b2_sep0_sc, b2_sep0_sh,
           b2_sep1_dw, b2_sep1_pw, b2_sep1_sc, b2_sep1_sh,
           b2_c22w, b2_c22b, b2_skw, b2_sksc, b2_sksh,
           mid_dw, mid_pw, mid_sc, mid_sh):
    B = x.shape[0]
    bw = (
        ((b0_sep0_dw, b0_sep0_pw, b0_sep0_sc, b0_sep0_sh),
         (b0_sep1_dw, b0_sep1_pw, b0_sep1_sc, b0_sep1_sh),
         b0_c22w, b0_c22b, b0_skw, b0_sksc, b0_sksh),
        ((b1_sep0_dw, b1_sep0_pw, b1_sep0_sc, b1_sep0_sh),
         (b1_sep1_dw, b1_sep1_pw, b1_sep1_sc, b1_sep1_sh),
         b1_c22w, b1_c22b, b1_skw, b1_sksc, b1_sksh),
        ((b2_sep0_dw, b2_sep0_pw, b2_sep0_sc, b2_sep0_sh),
         (b2_sep1_dw, b2_sep1_pw, b2_sep1_sc, b2_sep1_sh),
         b2_c22w, b2_c22b, b2_skw, b2_sksc, b2_sksh),
    )
    y = _encoder(x, conv1_w, bn1_scale, bn1_shift,
                 conv2_w, bn2_scale, bn2_shift, bw,
                 mid_dw, mid_pw, mid_sc, mid_sh)
    C = y.shape[-1]
    y = y.reshape(2, 2, B, C)
    return jnp.transpose(y, (2, 3, 0, 1))
